# Initial kernel scaffold; baseline (speedup 1.0000x reference)
import jax
import jax.numpy as jnp
from jax.experimental import pallas as pl


def kernel(x, edge_index, W):
    raise NotImplementedError("write your pallas kernel here")


# speedup vs baseline: 5.8758x; 5.8758x over previous
import functools

import jax
import jax.numpy as jnp
from jax import lax
from jax.experimental import pallas as pl
from jax.experimental.pallas import tpu as pltpu
from jax.experimental.pallas import tpu_sc as plsc

N_NODES = 10000
N_EDGES = 320000
D_IN = 128
D_HID = 128
D_PAD = D_IN + 16

NC = 2
NS = 16
NW = NC * NS
E_PER_TILE = N_EDGES // NW
K = 80
CHUNKS = E_PER_TILE // K
N_PAD = 10240
ROWS_PER_TILE = N_PAD // NS
ZROWS = 128


def _sc_aggregate(x_pad, ei_flat):
    mesh = plsc.VectorSubcoreMesh(core_axis_name="c", subcore_axis_name="s")

    @functools.partial(
        pl.kernel,
        out_type=jax.ShapeDtypeStruct((NC, N_PAD, D_PAD), jnp.float32),
        mesh=mesh,
        compiler_params=pltpu.CompilerParams(use_tc_tiling_on_sc=False),
        scratch_types=[
            pltpu.VMEM((K,), jnp.int32),
            pltpu.VMEM((K,), jnp.int32),
            pltpu.VMEM((K, D_PAD), jnp.float32),
            pltpu.VMEM((ZROWS, D_PAD), jnp.float32),
            pltpu.VMEM_SHARED((N_PAD, D_PAD), jnp.float32),
            pltpu.SemaphoreType.DMA,
        ],
    )
    def k(x_hbm, ei_hbm, part_hbm, src_v, dst_v, rows_v, zbuf, acc_s, sem):
        c = lax.axis_index("c")
        s = lax.axis_index("s")
        wid = c * NS + s

        @pl.loop(0, ZROWS)
        def _(r):
            for j in range(D_PAD // 16):
                zbuf[r, pl.ds(j * 16, 16)] = jnp.zeros((16,), jnp.float32)

        row0 = s * ROWS_PER_TILE
        @pl.loop(0, ROWS_PER_TILE // ZROWS)
        def _(b):
            pltpu.sync_copy(zbuf, acc_s.at[pl.ds(row0 + b * ZROWS, ZROWS)])

        plsc.subcore_barrier()

        ebase = wid * E_PER_TILE
        @pl.loop(0, CHUNKS)
        def _(i):
            base = ebase + i * K
            pltpu.sync_copy(ei_hbm.at[pl.ds(base, K)], src_v)
            pltpu.sync_copy(ei_hbm.at[pl.ds(N_EDGES + base, K)], dst_v)
            pltpu.async_copy(x_hbm.at[src_v], rows_v, sem).wait()
            pltpu.sync_copy(rows_v, acc_s.at[dst_v], add=True)

        plsc.subcore_barrier()

        @pl.loop(0, ROWS_PER_TILE // ZROWS)
        def _(b):
            r = row0 + b * ZROWS
            pltpu.sync_copy(acc_s.at[pl.ds(r, ZROWS)], zbuf)
            pltpu.sync_copy(zbuf, part_hbm.at[c, pl.ds(r, ZROWS)])

    return k(x_pad, ei_flat)


def _tc_finish_body(part_ref, w_ref, out_ref):
    ssum = part_ref[0, :N_NODES, :D_IN] + part_ref[1, :N_NODES, :D_IN]
    count = (part_ref[0, :N_NODES, D_IN:D_IN + 1]
             + part_ref[1, :N_NODES, D_IN:D_IN + 1])
    mean = ssum / jnp.maximum(count, 1.0)
    out_ref[...] = lax.dot_general(
        mean, w_ref[...], (((1,), (1,)), ((), ())),
        preferred_element_type=jnp.float32,
        precision=lax.Precision.HIGHEST,
    )


def _tc_finish(parts, W):
    return pl.pallas_call(
        _tc_finish_body,
        out_shape=jax.ShapeDtypeStruct((N_NODES, D_HID), jnp.float32),
    )(parts, W)


def kernel(x, edge_index, W):
    ei_flat = edge_index.reshape(-1)
    x_pad = jnp.concatenate(
        [x, jnp.ones((N_NODES, D_PAD - D_IN), jnp.float32)], axis=1)
    parts = _sc_aggregate(x_pad, ei_flat)
    return _tc_finish(parts, W)


# speedup vs baseline: 9.8499x; 1.6764x over previous
import functools

import jax
import jax.numpy as jnp
from jax import lax
from jax.experimental import pallas as pl
from jax.experimental.pallas import tpu as pltpu
from jax.experimental.pallas import tpu_sc as plsc

N_NODES = 10000
N_EDGES = 320000
D_IN = 128
D_HID = 128
D_PAD = D_IN + 16

NC = 2
NS = 16
NW = NC * NS
E_PER_TILE = N_EDGES // NW
K = 80
BLK = 2000
NBLK = E_PER_TILE // BLK
BCH = BLK // K
N_PAD = 10240
ROWS_PER_TILE = N_PAD // NS
ZROWS = 80


def _sc_aggregate(x_pad, ei_flat):
    mesh = plsc.VectorSubcoreMesh(core_axis_name="c", subcore_axis_name="s")

    @functools.partial(
        pl.kernel,
        out_type=jax.ShapeDtypeStruct((NC, N_PAD, D_PAD), jnp.float32),
        mesh=mesh,
        compiler_params=pltpu.CompilerParams(use_tc_tiling_on_sc=False),
        scratch_types=[
            [pltpu.VMEM((BLK,), jnp.int32) for _ in range(2)],
            [pltpu.VMEM((BLK,), jnp.int32) for _ in range(2)],
            [pltpu.VMEM((K, D_PAD), jnp.float32) for _ in range(2)],
            pltpu.VMEM_SHARED((N_PAD, D_PAD), jnp.float32),
            [pltpu.SemaphoreType.DMA for _ in range(2)],
            [pltpu.SemaphoreType.DMA for _ in range(2)],
        ],
    )
    def k(x_hbm, ei_hbm, part_hbm, srcb, dstb, rows, acc_s, gsem, isem):
        c = lax.axis_index("c")
        s = lax.axis_index("s")
        wid = c * NS + s
        ebase = wid * E_PER_TILE

        def load_block(j, p):
            pltpu.async_copy(
                ei_hbm.at[pl.ds(ebase + j * BLK, BLK)], srcb[p], isem[p])
            pltpu.async_copy(
                ei_hbm.at[pl.ds(N_EDGES + ebase + j * BLK, BLK)],
                dstb[p], isem[p])

        def wait_block(j, p):
            pltpu.make_async_copy(
                ei_hbm.at[pl.ds(ebase + j * BLK, BLK)], srcb[p], isem[p]).wait()
            pltpu.make_async_copy(
                ei_hbm.at[pl.ds(ebase + j * BLK, BLK)], dstb[p], isem[p]).wait()

        load_block(0, 0)

        @pl.loop(0, ZROWS)
        def _(r):
            for j in range(D_PAD // 16):
                rows[0][r, pl.ds(j * 16, 16)] = jnp.zeros((16,), jnp.float32)

        row0 = s * ROWS_PER_TILE
        @pl.loop(0, ROWS_PER_TILE // ZROWS)
        def _(b):
            pltpu.sync_copy(rows[0], acc_s.at[pl.ds(row0 + b * ZROWS, ZROWS)])

        plsc.subcore_barrier()

        def gather(p, sv, i):
            pltpu.async_copy(x_hbm.at[sv.at[pl.ds(i * K, K)]], rows[p], gsem[p])

        def gwait(p):
            pltpu.make_async_copy(x_hbm.at[srcb[0].at[pl.ds(0, K)]],
                                  rows[p], gsem[p]).wait()

        def scatter(p, dv, i):
            pltpu.sync_copy(rows[p], acc_s.at[dv.at[pl.ds(i * K, K)]], add=True)

        for j in range(NBLK):
            p = j % 2
            sv, dv = srcb[p], dstb[p]
            wait_block(j, p)
            if j + 1 < NBLK:
                load_block(j + 1, 1 - p)
            gather(0, sv, 0)

            @pl.loop(0, (BCH - 1) // 2)
            def _(g):
                i0 = g * 2
                gwait(0)
                gather(1, sv, i0 + 1)
                scatter(0, dv, i0)
                gwait(1)
                gather(0, sv, i0 + 2)
                scatter(1, dv, i0 + 1)

            gwait(0)
            scatter(0, dv, BCH - 1)

        plsc.subcore_barrier()

        @pl.loop(0, ROWS_PER_TILE // ZROWS)
        def _(b):
            r = row0 + b * ZROWS
            pltpu.sync_copy(acc_s.at[pl.ds(r, ZROWS)], rows[0])
            pltpu.sync_copy(rows[0], part_hbm.at[c, pl.ds(r, ZROWS)])

    return k(x_pad, ei_flat)


def _tc_finish_body(part_ref, w_ref, out_ref):
    ssum = part_ref[0, :N_NODES, :D_IN] + part_ref[1, :N_NODES, :D_IN]
    count = (part_ref[0, :N_NODES, D_IN:D_IN + 1]
             + part_ref[1, :N_NODES, D_IN:D_IN + 1])
    mean = ssum / jnp.maximum(count, 1.0)
    out_ref[...] = lax.dot_general(
        mean, w_ref[...], (((1,), (1,)), ((), ())),
        preferred_element_type=jnp.float32,
        precision=lax.Precision.HIGHEST,
    )


def _tc_finish(parts, W):
    return pl.pallas_call(
        _tc_finish_body,
        out_shape=jax.ShapeDtypeStruct((N_NODES, D_HID), jnp.float32),
    )(parts, W)


def kernel(x, edge_index, W):
    ei_flat = edge_index.reshape(-1)
    x_pad = jnp.concatenate(
        [x, jnp.ones((N_NODES, D_PAD - D_IN), jnp.float32)], axis=1)
    parts = _sc_aggregate(x_pad, ei_flat)
    return _tc_finish(parts, W)


# speedup vs baseline: 9.9896x; 1.0142x over previous
import functools

import jax
import jax.numpy as jnp
from jax import lax
from jax.experimental import pallas as pl
from jax.experimental.pallas import tpu as pltpu
from jax.experimental.pallas import tpu_sc as plsc

N_NODES = 10000
N_EDGES = 320000
D_IN = 128
D_HID = 128
D_PAD = D_IN + 16

NC = 2
NS = 16
NW = NC * NS
E_PER_TILE = N_EDGES // NW
K = 80
BLK = 2000
NBLK = E_PER_TILE // BLK
BCH = BLK // K
N_PAD = 10240
ROWS_PER_TILE = N_PAD // NS
ZROWS = 80


def _sc_aggregate(x_pad, ei_flat):
    mesh = plsc.VectorSubcoreMesh(core_axis_name="c", subcore_axis_name="s")

    @functools.partial(
        pl.kernel,
        out_type=jax.ShapeDtypeStruct((NC, N_PAD, D_PAD), jnp.float32),
        mesh=mesh,
        compiler_params=pltpu.CompilerParams(use_tc_tiling_on_sc=False),
        scratch_types=[
            [pltpu.VMEM((BLK,), jnp.int32) for _ in range(2)],
            [pltpu.VMEM((BLK,), jnp.int32) for _ in range(2)],
            [pltpu.VMEM((K, D_PAD), jnp.float32) for _ in range(2)],
            pltpu.VMEM_SHARED((N_PAD, D_PAD), jnp.float32),
            [pltpu.SemaphoreType.DMA for _ in range(2)],
            [pltpu.SemaphoreType.DMA for _ in range(2)],
        ],
    )
    def k(x_hbm, ei_hbm, part_hbm, srcb, dstb, rows, acc_s, gsem, isem):
        c = lax.axis_index("c")
        s = lax.axis_index("s")
        wid = c * NS + s
        ebase = wid * E_PER_TILE

        def load_block(j, p):
            pltpu.async_copy(
                ei_hbm.at[pl.ds(ebase + j * BLK, BLK)], srcb[p], isem[p])
            pltpu.async_copy(
                ei_hbm.at[pl.ds(N_EDGES + ebase + j * BLK, BLK)],
                dstb[p], isem[p])

        def wait_block(j, p):
            pltpu.make_async_copy(
                ei_hbm.at[pl.ds(ebase + j * BLK, BLK)], srcb[p], isem[p]).wait()
            pltpu.make_async_copy(
                ei_hbm.at[pl.ds(ebase + j * BLK, BLK)], dstb[p], isem[p]).wait()

        load_block(0, 0)

        @pl.loop(0, ZROWS)
        def _(r):
            for j in range(D_PAD // 16):
                rows[0][r, pl.ds(j * 16, 16)] = jnp.zeros((16,), jnp.float32)

        row0 = s * ROWS_PER_TILE
        @pl.loop(0, ROWS_PER_TILE // ZROWS)
        def _(b):
            pltpu.sync_copy(rows[0], acc_s.at[pl.ds(row0 + b * ZROWS, ZROWS)])

        plsc.subcore_barrier()

        def gather(p, sv, i):
            pltpu.async_copy(x_hbm.at[sv.at[pl.ds(i * K, K)]], rows[p], gsem[p])

        def gwait(p):
            pltpu.make_async_copy(x_hbm.at[srcb[0].at[pl.ds(0, K)]],
                                  rows[p], gsem[p]).wait()

        def scatter(p, dv, i):
            pltpu.sync_copy(rows[p], acc_s.at[dv.at[pl.ds(i * K, K)]], add=True)

        for j in range(NBLK):
            p = j % 2
            sv, dv = srcb[p], dstb[p]
            wait_block(j, p)
            if j + 1 < NBLK:
                load_block(j + 1, 1 - p)
            gather(0, sv, 0)

            @pl.loop(0, (BCH - 1) // 2)
            def _(g):
                i0 = g * 2
                gwait(0)
                gather(1, sv, i0 + 1)
                scatter(0, dv, i0)
                gwait(1)
                gather(0, sv, i0 + 2)
                scatter(1, dv, i0 + 1)

            gwait(0)
            scatter(0, dv, BCH - 1)

        plsc.subcore_barrier()

        @pl.loop(0, ROWS_PER_TILE // ZROWS)
        def _(b):
            r = row0 + b * ZROWS
            pltpu.sync_copy(acc_s.at[pl.ds(r, ZROWS)], rows[0])
            pltpu.sync_copy(rows[0], part_hbm.at[c, pl.ds(r, ZROWS)])

    return k(x_pad, ei_flat)


def _tc_finish_body(part_ref, w_ref, out_ref):
    ssum = part_ref[0, :N_NODES, :D_IN] + part_ref[1, :N_NODES, :D_IN]
    count = (part_ref[0, :N_NODES, D_IN:D_IN + 1]
             + part_ref[1, :N_NODES, D_IN:D_IN + 1])
    mean = ssum / jnp.maximum(count, 1.0)
    out_ref[...] = lax.dot_general(
        mean, w_ref[...], (((1,), (1,)), ((), ())),
        preferred_element_type=jnp.float32,
    )


def _tc_finish(parts, W):
    return pl.pallas_call(
        _tc_finish_body,
        out_shape=jax.ShapeDtypeStruct((N_NODES, D_HID), jnp.float32),
    )(parts, W)


def kernel(x, edge_index, W):
    ei_flat = edge_index.reshape(-1)
    x_pad = jnp.concatenate(
        [x, jnp.ones((N_NODES, D_PAD - D_IN), jnp.float32)], axis=1)
    parts = _sc_aggregate(x_pad, ei_flat)
    return _tc_finish(parts, W)


# speedup vs baseline: 10.0397x; 1.0050x over previous
import functools

import jax
import jax.numpy as jnp
from jax import lax
from jax.experimental import pallas as pl
from jax.experimental.pallas import tpu as pltpu
from jax.experimental.pallas import tpu_sc as plsc

N_NODES = 10000
N_EDGES = 320000
D_IN = 128
D_HID = 128
D_PAD = D_IN + 16

NC = 2
NS = 16
NW = NC * NS
E_PER_TILE = N_EDGES // NW
K = 80
BLK = 2000
NBLK = E_PER_TILE // BLK
BCH = BLK // K
N_PAD = 10240
ROWS_PER_TILE = N_PAD // NS
ZROWS = 80


def _sc_aggregate(x_pad, ei_flat):
    mesh = plsc.VectorSubcoreMesh(core_axis_name="c", subcore_axis_name="s")

    @functools.partial(
        pl.kernel,
        out_type=jax.ShapeDtypeStruct((NC, N_PAD, D_PAD), jnp.float32),
        mesh=mesh,
        compiler_params=pltpu.CompilerParams(use_tc_tiling_on_sc=False),
        scratch_types=[
            [pltpu.VMEM((BLK,), jnp.int32) for _ in range(2)],
            [pltpu.VMEM((BLK,), jnp.int32) for _ in range(2)],
            [pltpu.VMEM((K, D_PAD), jnp.float32) for _ in range(2)],
            pltpu.VMEM_SHARED((N_PAD, D_PAD), jnp.float32),
            [pltpu.SemaphoreType.DMA for _ in range(2)],
            [pltpu.SemaphoreType.DMA for _ in range(2)],
            [pltpu.SemaphoreType.DMA for _ in range(2)],
        ],
    )
    def k(x_hbm, ei_hbm, part_hbm, srcb, dstb, rows, acc_s, gsem, ssem, isem):
        c = lax.axis_index("c")
        s = lax.axis_index("s")
        wid = c * NS + s
        ebase = wid * E_PER_TILE

        def load_block(j, p):
            pltpu.async_copy(
                ei_hbm.at[pl.ds(ebase + j * BLK, BLK)], srcb[p], isem[p])
            pltpu.async_copy(
                ei_hbm.at[pl.ds(N_EDGES + ebase + j * BLK, BLK)],
                dstb[p], isem[p])

        def wait_block(j, p):
            pltpu.make_async_copy(
                ei_hbm.at[pl.ds(ebase + j * BLK, BLK)], srcb[p], isem[p]).wait()
            pltpu.make_async_copy(
                ei_hbm.at[pl.ds(ebase + j * BLK, BLK)], dstb[p], isem[p]).wait()

        load_block(0, 0)

        @pl.loop(0, ZROWS)
        def _(r):
            for j in range(D_PAD // 16):
                rows[0][r, pl.ds(j * 16, 16)] = jnp.zeros((16,), jnp.float32)

        row0 = s * ROWS_PER_TILE
        NZC = ROWS_PER_TILE // ZROWS
        @pl.loop(0, NZC)
        def _(b):
            pltpu.async_copy(rows[0], acc_s.at[pl.ds(row0 + b * ZROWS, ZROWS)],
                             ssem[0])

        @pl.loop(0, NZC)
        def _(b):
            pltpu.make_async_copy(
                rows[0], acc_s.at[pl.ds(row0, ZROWS)], ssem[0]).wait()

        plsc.subcore_barrier()

        def gather(p, sv, i):
            pltpu.async_copy(x_hbm.at[sv.at[pl.ds(i * K, K)]], rows[p], gsem[p])

        def gwait(p):
            pltpu.make_async_copy(x_hbm.at[srcb[0].at[pl.ds(0, K)]],
                                  rows[p], gsem[p]).wait()

        def scatter(p, dv, i):
            pltpu.async_copy(rows[p], acc_s.at[dv.at[pl.ds(i * K, K)]],
                             ssem[p], add=True)

        def swait(p):
            pltpu.make_async_copy(
                rows[p], acc_s.at[dstb[0].at[pl.ds(0, K)]], ssem[p]).wait()

        for j in range(NBLK):
            p = j % 2
            sv, dv = srcb[p], dstb[p]
            wait_block(j, p)
            if j + 1 < NBLK:
                load_block(j + 1, 1 - p)

            gather(0, sv, 0)
            gwait(0)
            scatter(0, dv, 0)
            gather(1, sv, 1)

            @pl.loop(0, (BCH - 2) // 2)
            def _(g):
                i0 = 1 + g * 2
                gwait(1)
                scatter(1, dv, i0)
                swait(0)
                gather(0, sv, i0 + 1)
                gwait(0)
                scatter(0, dv, i0 + 1)
                swait(1)
                gather(1, sv, i0 + 2)

            gwait(1)
            scatter(1, dv, BCH - 2)
            swait(0)
            gather(0, sv, BCH - 1)
            gwait(0)
            scatter(0, dv, BCH - 1)
            swait(1)
            swait(0)

        plsc.subcore_barrier()

        @pl.loop(0, NZC // 2)
        def _(b):
            r0 = row0 + (2 * b) * ZROWS
            r1 = row0 + (2 * b + 1) * ZROWS
            pltpu.sync_copy(acc_s.at[pl.ds(r0, ZROWS)], rows[0])
            pltpu.async_copy(rows[0], part_hbm.at[c, pl.ds(r0, ZROWS)], gsem[0])
            pltpu.sync_copy(acc_s.at[pl.ds(r1, ZROWS)], rows[1])
            pltpu.async_copy(rows[1], part_hbm.at[c, pl.ds(r1, ZROWS)], gsem[1])
            pltpu.make_async_copy(
                rows[0], part_hbm.at[c, pl.ds(r0, ZROWS)], gsem[0]).wait()
            pltpu.make_async_copy(
                rows[1], part_hbm.at[c, pl.ds(r1, ZROWS)], gsem[1]).wait()

    return k(x_pad, ei_flat)


def _tc_finish_body(part_ref, w_ref, out_ref):
    ssum = part_ref[0, :N_NODES, :D_IN] + part_ref[1, :N_NODES, :D_IN]
    count = (part_ref[0, :N_NODES, D_IN:D_IN + 1]
             + part_ref[1, :N_NODES, D_IN:D_IN + 1])
    mean = ssum / jnp.maximum(count, 1.0)
    out_ref[...] = lax.dot_general(
        mean, w_ref[...], (((1,), (1,)), ((), ())),
        preferred_element_type=jnp.float32,
    )


def _tc_finish(parts, W):
    return pl.pallas_call(
        _tc_finish_body,
        out_shape=jax.ShapeDtypeStruct((N_NODES, D_HID), jnp.float32),
    )(parts, W)


def kernel(x, edge_index, W):
    ei_flat = edge_index.reshape(-1)
    x_pad = jnp.concatenate(
        [x, jnp.ones((N_NODES, D_PAD - D_IN), jnp.float32)], axis=1)
    parts = _sc_aggregate(x_pad, ei_flat)
    return _tc_finish(parts, W)
